# baseline (speedup 1.0000x reference)
import jax
import jax.numpy as jnp
from jax.experimental import pallas as pl
from jax.experimental.pallas import tpu as pltpu

K = 512


def _tap_masks(nlanes, h, w):
    lane = jax.lax.broadcasted_iota(jnp.int32, (1, nlanes), 1)
    s = lane % (h * w)
    return s // w, s % w


def _conv1_pool_body(x_ref, w_ref, b_ref, o_ref):
    xf = x_ref[...]
    n = xf.shape[1]
    i, j = _tap_masks(n, 28, 28)
    rows = []
    for k in range(25):
        di, dj = k // 5 - 2, k % 5 - 2
        r = di * 28 + dj
        z = pltpu.roll(xf, (-r) % n, axis=1)
        ok = ((i + di >= 0) & (i + di < 28) & (j + dj >= 0) & (j + dj < 28))
        rows.append(z * ok.astype(jnp.float32))
    patches = jnp.concatenate(rows, axis=0)
    y = jnp.dot(w_ref[...], patches)
    y = jnp.maximum(y + b_ref[...], 0.0)
    m = jnp.maximum(y, pltpu.roll(y, (-1) % n, axis=1))
    m = jnp.maximum(m, pltpu.roll(m, (-28) % n, axis=1))
    o_ref[...] = m


def _conv1_pool(xf, w1, b1):
    nb = 8
    nl = 100352 // nb
    return pl.pallas_call(
        _conv1_pool_body,
        grid=(nb,),
        in_specs=[
            pl.BlockSpec((1, nl), lambda g: (0, g)),
            pl.BlockSpec((32, 25), lambda g: (0, 0)),
            pl.BlockSpec((32, 1), lambda g: (0, 0)),
        ],
        out_specs=pl.BlockSpec((32, nl), lambda g: (0, g)),
        out_shape=jax.ShapeDtypeStruct((32, 100352), jnp.float32),
    )(xf, w1, b1)


def _conv_mid_acc(hw, cout, x_ref, w_ref, b_ref):
    X = x_ref[...]
    n = X.shape[1]
    h = w = hw
    i, j = _tap_masks(n, h, w)
    acc = jnp.zeros((cout, n), jnp.float32)
    for k in range(25):
        di, dj = k // 5 - 2, k % 5 - 2
        r = di * w + dj
        z = pltpu.roll(X, (-r) % n, axis=1)
        ok = ((i + di >= 0) & (i + di < h) & (j + dj >= 0) & (j + dj < w))
        z = z * ok.astype(jnp.float32)
        acc = acc + jnp.dot(w_ref[k], z)
    return jnp.maximum(acc + b_ref[...], 0.0)


def _conv2(X, w, b):
    nb = 4
    nl = 25088 // nb
    def body(x_ref, w_ref, b_ref, o_ref):
        o_ref[...] = _conv_mid_acc(14, 32, x_ref, w_ref, b_ref)
    return pl.pallas_call(
        body,
        grid=(nb,),
        in_specs=[
            pl.BlockSpec((32, nl), lambda g: (0, g)),
            pl.BlockSpec((25, 32, 32), lambda g: (0, 0, 0)),
            pl.BlockSpec((32, 1), lambda g: (0, 0)),
        ],
        out_specs=pl.BlockSpec((32, nl), lambda g: (0, g)),
        out_shape=jax.ShapeDtypeStruct((32, 25088), jnp.float32),
    )(X, w, b)


def _conv3_pool_body(x_ref, w_ref, b_ref, o_ref):
    y = _conv_mid_acc(14, 64, x_ref, w_ref, b_ref)
    n = y.shape[1]
    m = jnp.maximum(y, pltpu.roll(y, (-1) % n, axis=1))
    m = jnp.maximum(m, pltpu.roll(m, (-14) % n, axis=1))
    o_ref[...] = m


def _conv3_pool(X, w, b):
    nb = 4
    nl = 25088 // nb
    return pl.pallas_call(
        _conv3_pool_body,
        grid=(nb,),
        in_specs=[
            pl.BlockSpec((32, nl), lambda g: (0, g)),
            pl.BlockSpec((25, 64, 32), lambda g: (0, 0, 0)),
            pl.BlockSpec((64, 1), lambda g: (0, 0)),
        ],
        out_specs=pl.BlockSpec((64, nl), lambda g: (0, g)),
        out_shape=jax.ShapeDtypeStruct((64, 25088), jnp.float32),
    )(X, w, b)


def _knn_body(x_ref, c_ref, o_ref):
    xf = x_ref[0]
    cb = c_ref[0]
    n = jnp.sqrt(jnp.sum(xf * xf, axis=1, keepdims=True))
    xn = xf / jnp.maximum(n, 1e-12)
    d = 1.0 - jax.lax.dot_general(xn, cb, (((1,), (1,)), ((), ())))
    dmin = jnp.min(d, axis=1, keepdims=True)
    ii = jax.lax.broadcasted_iota(jnp.int32, d.shape, 1)
    idx = jnp.min(jnp.where(d == dmin, ii, K), axis=1, keepdims=True)
    onehot = (ii == idx).astype(jnp.float32)
    o_ref[0] = jnp.dot(onehot, cb)


def _knn(h, cb):
    C, B, D = h.shape
    return pl.pallas_call(
        _knn_body,
        grid=(C,),
        in_specs=[
            pl.BlockSpec((1, B, D), lambda c: (c, 0, 0)),
            pl.BlockSpec((1, K, D), lambda c: (c, 0, 0)),
        ],
        out_specs=pl.BlockSpec((1, B, D), lambda c: (c, 0, 0)),
        out_shape=jax.ShapeDtypeStruct((C, B, D), jnp.float32),
    )(h, cb)


def _fc_body(x_ref, w1_ref, b1_ref, w2_ref, b2_ref, o_ref):
    h = jax.lax.dot_general(x_ref[...], w1_ref[...], (((1,), (1,)), ((), ())))
    h = jnp.maximum(h + b1_ref[...], 0.0)
    o = jax.lax.dot_general(h, w2_ref[...], (((1,), (1,)), ((), ())))
    o_ref[...] = o + b2_ref[...]


def _fc(x, w1, b1, w2, b2):
    return pl.pallas_call(
        _fc_body,
        grid=(1,),
        in_specs=[
            pl.BlockSpec((128, 3136), lambda i: (0, 0)),
            pl.BlockSpec((1024, 3136), lambda i: (0, 0)),
            pl.BlockSpec((1, 1024), lambda i: (0, 0)),
            pl.BlockSpec((10, 1024), lambda i: (0, 0)),
            pl.BlockSpec((1, 10), lambda i: (0, 0)),
        ],
        out_specs=pl.BlockSpec((128, 10), lambda i: (0, 0)),
        out_shape=jax.ShapeDtypeStruct((128, 10), jnp.float32),
    )(x, w1, b1, w2, b2)


def kernel(x, conv1_w, conv1_b, conv2_w, conv2_b, conv3_w, conv3_b,
           fc1_w, fc1_b, fc2_w, fc2_b, c0, c1, c2):
    xf = x.reshape(1, 128 * 784)
    h = _conv1_pool(xf, conv1_w.reshape(32, 25), conv1_b.reshape(32, 1))
    h = h.reshape(32, 128, 28, 28)[:, :, ::2, ::2].reshape(32, 128, 196)
    h = _knn(h, c0)

    w2 = jnp.transpose(conv2_w.reshape(32, 32, 25), (2, 0, 1))
    h = _conv2(h.reshape(32, 25088), w2, conv2_b.reshape(32, 1))
    h = _knn(h.reshape(32, 128, 196), c1)

    w3 = jnp.transpose(conv3_w.reshape(64, 32, 25), (2, 0, 1))
    h = _conv3_pool(h.reshape(32, 25088), w3, conv3_b.reshape(64, 1))
    h = h.reshape(64, 128, 14, 14)[:, :, ::2, ::2].reshape(64, 128, 49)
    h = _knn(h, c2)

    hf = jnp.transpose(h, (1, 0, 2)).reshape(128, 3136)
    return _fc(hf, fc1_w, fc1_b.reshape(1, 1024),
               fc2_w, fc2_b.reshape(1, 10))
